# initial kernel scaffold (unmeasured)
import jax
import jax.numpy as jnp
from jax import lax
from jax.experimental import pallas as pl
from jax.experimental.pallas import tpu as pltpu

N_DEV = 4
SQ = 2048
DM = 1024
H = 8
DH = 128
BLK = 64
R = 4
JB = 8
CHUNK = SQ // N_DEV
SCALE = 0.08838834764831843


def _body(x_hbm, wq_ref, k_hbm, v_hbm, wo_ref, out_ref,
          xr, qr, kr, vr, o_buf, stats_r, stats_comm,
          dma_sems, st_send, st_recv,
          rs_comm, rs_send, rs_recv, ag_send, ag_recv):
    my = lax.axis_index("i")
    right = jnp.mod(my + 1, N_DEV)
    left = jnp.mod(my - 1, N_DEV)

    for r in range(R):
        blocks = [r + R * j for j in range(JB)]
        copies = []
        for j, b in enumerate(blocks):
            for src, dst, si in ((x_hbm, xr, 0), (k_hbm, kr, 1), (v_hbm, vr, 2)):
                c = pltpu.make_async_copy(
                    src.at[pl.ds(b * BLK, BLK), :],
                    dst.at[pl.ds(j * BLK, BLK), :],
                    dma_sems.at[si],
                )
                c.start()
                copies.append(c)
        for c in copies:
            c.wait()

        qr[:, :] = jnp.dot(xr[:, :], wq_ref[:, :],
                           preferred_element_type=jnp.float32)
        for h in range(H):
            cs = slice(h * DH, (h + 1) * DH)
            s = lax.dot_general(qr[:, cs], kr[:, cs],
                                (((1,), (1,)), ((), ())),
                                preferred_element_type=jnp.float32) * SCALE
            m = jnp.max(s, axis=1, keepdims=True)
            w = jnp.exp(s - m)
            l = jnp.sum(w, axis=1, keepdims=True)
            o = jnp.dot(w, vr[:, cs], preferred_element_type=jnp.float32)
            for j, b in enumerate(blocks):
                o_buf[pl.ds(b * BLK, BLK), cs] = o[j * BLK:(j + 1) * BLK, :]
            stats_r[:, h:h + 1] = m
            stats_r[:, H + h:H + h + 1] = l
        for j, b in enumerate(blocks):
            stats_comm[0, pl.ds(b * BLK, BLK), :] = stats_r[pl.ds(j * BLK, BLK), :]

    barrier = pltpu.get_barrier_semaphore()
    for nbr in (left, right):
        pl.semaphore_signal(barrier, inc=1, device_id=(nbr,),
                            device_id_type=pl.DeviceIdType.MESH)
    pl.semaphore_wait(barrier, 2)

    for t in range(N_DEV - 1):
        rdma = pltpu.make_async_remote_copy(
            src_ref=stats_comm.at[t],
            dst_ref=stats_comm.at[t + 1],
            send_sem=st_send.at[t],
            recv_sem=st_recv.at[t],
            device_id=(right,),
            device_id_type=pl.DeviceIdType.MESH,
        )
        rdma.start()
        rdma.wait()

    m_all = [stats_comm[s, :, 0:H] for s in range(N_DEV)]
    l_all = [stats_comm[s, :, H:2 * H] for s in range(N_DEV)]
    M = jnp.maximum(jnp.maximum(m_all[0], m_all[1]),
                    jnp.maximum(m_all[2], m_all[3]))
    L = (jnp.exp(m_all[0] - M) * l_all[0] + jnp.exp(m_all[1] - M) * l_all[1]
         + jnp.exp(m_all[2] - M) * l_all[2] + jnp.exp(m_all[3] - M) * l_all[3])
    alpha = jnp.exp(m_all[0] - M) / L
    for h in range(H):
        cs = slice(h * DH, (h + 1) * DH)
        o_buf[:, cs] = o_buf[:, cs] * alpha[:, h:h + 1]

    out_ref[:, :] = jnp.dot(o_buf[:, :], wo_ref[:, :],
                            preferred_element_type=jnp.float32)

    def rows(idx):
        return pl.ds(idx * CHUNK, CHUNK)

    for t in range(N_DEV - 1):
        send_idx = jnp.mod(my - t, N_DEV)
        rdma = pltpu.make_async_remote_copy(
            src_ref=out_ref.at[rows(send_idx), :],
            dst_ref=rs_comm.at[t],
            send_sem=rs_send.at[t],
            recv_sem=rs_recv.at[t],
            device_id=(right,),
            device_id_type=pl.DeviceIdType.MESH,
        )
        rdma.start()
        rdma.wait()
        add_idx = jnp.mod(my - t - 1, N_DEV)
        out_ref[rows(add_idx), :] = out_ref[rows(add_idx), :] + rs_comm[t, :, :]

    for t in range(N_DEV - 1):
        fwd_idx = jnp.mod(my + 1 - t, N_DEV)
        rdma = pltpu.make_async_remote_copy(
            src_ref=out_ref.at[rows(fwd_idx), :],
            dst_ref=out_ref.at[rows(fwd_idx), :],
            send_sem=ag_send.at[t],
            recv_sem=ag_recv.at[t],
            device_id=(right,),
            device_id_type=pl.DeviceIdType.MESH,
        )
        rdma.start()
        rdma.wait()


def kernel(x, Wq, K_ext, V_ext, Wo):
    x2 = x.reshape(SQ, DM)
    k2 = K_ext.reshape(SQ, H * DH)
    v2 = V_ext.reshape(SQ, H * DH)
    out = pl.pallas_call(
        _body,
        out_shape=jax.ShapeDtypeStruct((SQ, DM), jnp.float32),
        in_specs=[
            pl.BlockSpec(memory_space=pltpu.ANY),
            pl.BlockSpec(memory_space=pltpu.VMEM),
            pl.BlockSpec(memory_space=pltpu.ANY),
            pl.BlockSpec(memory_space=pltpu.ANY),
            pl.BlockSpec(memory_space=pltpu.VMEM),
        ],
        out_specs=pl.BlockSpec(memory_space=pltpu.VMEM),
        scratch_shapes=[
            pltpu.VMEM((CHUNK, DM), jnp.float32),
            pltpu.VMEM((CHUNK, DM), jnp.float32),
            pltpu.VMEM((CHUNK, H * DH), jnp.float32),
            pltpu.VMEM((CHUNK, H * DH), jnp.float32),
            pltpu.VMEM((SQ, DM), jnp.float32),
            pltpu.VMEM((CHUNK, 2 * H), jnp.float32),
            pltpu.VMEM((N_DEV, SQ, 2 * H), jnp.float32),
            pltpu.SemaphoreType.DMA((3,)),
            pltpu.SemaphoreType.DMA((N_DEV - 1,)),
            pltpu.SemaphoreType.DMA((N_DEV - 1,)),
            pltpu.VMEM((N_DEV - 1, CHUNK, DM), jnp.float32),
            pltpu.SemaphoreType.DMA((N_DEV - 1,)),
            pltpu.SemaphoreType.DMA((N_DEV - 1,)),
            pltpu.SemaphoreType.DMA((N_DEV - 1,)),
            pltpu.SemaphoreType.DMA((N_DEV - 1,)),
        ],
        compiler_params=pltpu.CompilerParams(collective_id=0),
    )(x2, Wq, k2, v2, Wo)
    return out.reshape(1, SQ, DM)


# baseline (device time: 242994 ns/iter reference)
import jax
import jax.numpy as jnp
from jax import lax
from jax.experimental import pallas as pl
from jax.experimental.pallas import tpu as pltpu

N_DEV = 4
SQ = 2048
DM = 1024
H = 8
DH = 128
BLK = 64
R = 4
JB = 8
CHUNK = SQ // N_DEV
SCALE = 0.08838834764831843


def _body(x_hbm, wq_ref, k_hbm, v_hbm, wo_ref, out_ref,
          xr, qr, kr, vr, o_buf, stats_r, stats_comm,
          dma_sems, st_send, st_recv,
          rs_comm, rs_send, rs_recv, ag_send, ag_recv):
    my = lax.axis_index("i")
    right = jnp.mod(my + 1, N_DEV)
    left = jnp.mod(my - 1, N_DEV)

    for r in range(R):
        blocks = [r + R * j for j in range(JB)]
        copies = []
        for j, b in enumerate(blocks):
            for src, dst, si in ((x_hbm, xr, 0), (k_hbm, kr, 1), (v_hbm, vr, 2)):
                c = pltpu.make_async_copy(
                    src.at[pl.ds(b * BLK, BLK), :],
                    dst.at[pl.ds(j * BLK, BLK), :],
                    dma_sems.at[si],
                )
                c.start()
                copies.append(c)
        for c in copies:
            c.wait()

        qr[:, :] = jnp.dot(xr[:, :], wq_ref[:, :],
                           preferred_element_type=jnp.float32)
        for h in range(H):
            cs = slice(h * DH, (h + 1) * DH)
            s = lax.dot_general(qr[:, cs], kr[:, cs],
                                (((1,), (1,)), ((), ())),
                                preferred_element_type=jnp.float32) * SCALE
            m = jnp.max(s, axis=1, keepdims=True)
            w = jnp.exp(s - m)
            l = jnp.sum(w, axis=1, keepdims=True)
            o = jnp.dot(w, vr[:, cs], preferred_element_type=jnp.float32)
            for j, b in enumerate(blocks):
                o_buf[pl.ds(b * BLK, BLK), cs] = o[j * BLK:(j + 1) * BLK, :]
            stats_r[:, h:h + 1] = m
            stats_r[:, H + h:H + h + 1] = l
        for j, b in enumerate(blocks):
            stats_comm[0, pl.ds(b * BLK, BLK), :] = stats_r[pl.ds(j * BLK, BLK), :]

    barrier = pltpu.get_barrier_semaphore()
    for nbr in (left, right):
        pl.semaphore_signal(barrier, inc=1, device_id=(nbr,),
                            device_id_type=pl.DeviceIdType.MESH)
    pl.semaphore_wait(barrier, 2)

    for t in range(N_DEV - 1):
        rdma = pltpu.make_async_remote_copy(
            src_ref=stats_comm.at[t],
            dst_ref=stats_comm.at[t + 1],
            send_sem=st_send.at[t],
            recv_sem=st_recv.at[t],
            device_id=(right,),
            device_id_type=pl.DeviceIdType.MESH,
        )
        rdma.start()
        rdma.wait()

    m_all = [stats_comm[s, :, 0:H] for s in range(N_DEV)]
    l_all = [stats_comm[s, :, H:2 * H] for s in range(N_DEV)]
    M = jnp.maximum(jnp.maximum(m_all[0], m_all[1]),
                    jnp.maximum(m_all[2], m_all[3]))
    L = (jnp.exp(m_all[0] - M) * l_all[0] + jnp.exp(m_all[1] - M) * l_all[1]
         + jnp.exp(m_all[2] - M) * l_all[2] + jnp.exp(m_all[3] - M) * l_all[3])
    alpha = jnp.exp(m_all[0] - M) / L
    for h in range(H):
        cs = slice(h * DH, (h + 1) * DH)
        o_buf[:, cs] = o_buf[:, cs] * alpha[:, h:h + 1]

    out_ref[:, :] = jnp.dot(o_buf[:, :], wo_ref[:, :],
                            preferred_element_type=jnp.float32)

    def rows(idx):
        return pl.ds(idx * CHUNK, CHUNK)

    for t in range(N_DEV - 1):
        send_idx = jnp.mod(my - t, N_DEV)
        rdma = pltpu.make_async_remote_copy(
            src_ref=out_ref.at[rows(send_idx), :],
            dst_ref=rs_comm.at[t],
            send_sem=rs_send.at[t],
            recv_sem=rs_recv.at[t],
            device_id=(right,),
            device_id_type=pl.DeviceIdType.MESH,
        )
        rdma.start()
        rdma.wait()
        add_idx = jnp.mod(my - t - 1, N_DEV)
        out_ref[rows(add_idx), :] = out_ref[rows(add_idx), :] + rs_comm[t, :, :]

    for t in range(N_DEV - 1):
        fwd_idx = jnp.mod(my + 1 - t, N_DEV)
        rdma = pltpu.make_async_remote_copy(
            src_ref=out_ref.at[rows(fwd_idx), :],
            dst_ref=out_ref.at[rows(fwd_idx), :],
            send_sem=ag_send.at[t],
            recv_sem=ag_recv.at[t],
            device_id=(right,),
            device_id_type=pl.DeviceIdType.MESH,
        )
        rdma.start()
        rdma.wait()


def kernel(x, Wq, K_ext, V_ext, Wo):
    x2 = x.reshape(SQ, DM)
    k2 = K_ext.reshape(SQ, H * DH)
    v2 = V_ext.reshape(SQ, H * DH)
    out = pl.pallas_call(
        _body,
        out_shape=jax.ShapeDtypeStruct((SQ, DM), jnp.float32),
        in_specs=[
            pl.BlockSpec(memory_space=pl.ANY),
            pl.BlockSpec(memory_space=pltpu.VMEM),
            pl.BlockSpec(memory_space=pl.ANY),
            pl.BlockSpec(memory_space=pl.ANY),
            pl.BlockSpec(memory_space=pltpu.VMEM),
        ],
        out_specs=pl.BlockSpec(memory_space=pltpu.VMEM),
        scratch_shapes=[
            pltpu.VMEM((CHUNK, DM), jnp.float32),
            pltpu.VMEM((CHUNK, DM), jnp.float32),
            pltpu.VMEM((CHUNK, H * DH), jnp.float32),
            pltpu.VMEM((CHUNK, H * DH), jnp.float32),
            pltpu.VMEM((SQ, DM), jnp.float32),
            pltpu.VMEM((CHUNK, 2 * H), jnp.float32),
            pltpu.VMEM((N_DEV, SQ, 2 * H), jnp.float32),
            pltpu.SemaphoreType.DMA((3,)),
            pltpu.SemaphoreType.DMA((N_DEV - 1,)),
            pltpu.SemaphoreType.DMA((N_DEV - 1,)),
            pltpu.VMEM((N_DEV - 1, CHUNK, DM), jnp.float32),
            pltpu.SemaphoreType.DMA((N_DEV - 1,)),
            pltpu.SemaphoreType.DMA((N_DEV - 1,)),
            pltpu.SemaphoreType.DMA((N_DEV - 1,)),
            pltpu.SemaphoreType.DMA((N_DEV - 1,)),
        ],
        compiler_params=pltpu.CompilerParams(collective_id=0),
    )(x2, Wq, k2, v2, Wo)
    return out.reshape(1, SQ, DM)


# device time: 159457 ns/iter; 1.5239x vs baseline; 1.5239x over previous
import jax
import jax.numpy as jnp
from jax import lax
from jax.experimental import pallas as pl
from jax.experimental.pallas import tpu as pltpu

N_DEV = 4
SQ = 2048
DM = 1024
H = 8
DH = 128
BLK = 64
R = 4
JB = 8
CHUNK = SQ // N_DEV
HALF = DM // 2
SCALE = 0.08838834764831843
MESH = pl.DeviceIdType.MESH


def _body(x_hbm, wq_ref, k_hbm, v_hbm, wo_ref, out_ref,
          xr, qr, kr, vr, o_buf, stats_r, stats_comm,
          dma_sems, st_send, st_recv, send_cw, send_ccw,
          rs_cw, rs_ccw, rs_recv_cw, rs_recv_ccw, ag_recv_cw, ag_recv_ccw):
    my = lax.axis_index("i")
    right = jnp.mod(my + 1, N_DEV)
    left = jnp.mod(my - 1, N_DEV)
    diag = jnp.mod(my + 2, N_DEV)

    for r in range(R):
        blocks = [r + R * j for j in range(JB)]
        copies = []
        for j, b in enumerate(blocks):
            for src, dst, si in ((x_hbm, xr, 0), (k_hbm, kr, 1), (v_hbm, vr, 2)):
                c = pltpu.make_async_copy(
                    src.at[pl.ds(b * BLK, BLK), :],
                    dst.at[pl.ds(j * BLK, BLK), :],
                    dma_sems.at[si],
                )
                c.start()
                copies.append(c)
        for c in copies:
            c.wait()

        qr[:, :] = jnp.dot(xr[:, :], wq_ref[:, :],
                           preferred_element_type=jnp.float32)
        for h in range(H):
            cs = slice(h * DH, (h + 1) * DH)
            s = lax.dot_general(qr[:, cs], kr[:, cs],
                                (((1,), (1,)), ((), ())),
                                preferred_element_type=jnp.float32) * SCALE
            m = jnp.max(s, axis=1, keepdims=True)
            w = jnp.exp(s - m)
            l = jnp.sum(w, axis=1, keepdims=True)
            o = jnp.dot(w, vr[:, cs], preferred_element_type=jnp.float32)
            for j, b in enumerate(blocks):
                o_buf[pl.ds(b * BLK, BLK), cs] = o[j * BLK:(j + 1) * BLK, :]
            stats_r[:, h:h + 1] = m
            stats_r[:, H + h:H + h + 1] = l
        for j, b in enumerate(blocks):
            stats_comm[0, pl.ds(b * BLK, BLK), :] = stats_r[pl.ds(j * BLK, BLK), :]

    barrier = pltpu.get_barrier_semaphore()
    for nbr in (left, right, diag):
        pl.semaphore_signal(barrier, inc=1, device_id=(nbr,),
                            device_id_type=MESH)
    pl.semaphore_wait(barrier, 3)

    stat_sends = []
    for d in range(1, N_DEV):
        slot = N_DEV - d
        rdma = pltpu.make_async_remote_copy(
            src_ref=stats_comm.at[0],
            dst_ref=stats_comm.at[slot],
            send_sem=st_send.at[d - 1],
            recv_sem=st_recv.at[slot - 1],
            device_id=(jnp.mod(my + d, N_DEV),),
            device_id_type=MESH,
        )
        rdma.start()
        stat_sends.append(rdma)
    for k in range(1, N_DEV):
        pltpu.make_async_remote_copy(
            src_ref=stats_comm.at[k], dst_ref=stats_comm.at[k],
            send_sem=st_send.at[k - 1], recv_sem=st_recv.at[k - 1],
            device_id=(left,), device_id_type=MESH,
        ).wait_recv()
    for rdma in stat_sends:
        rdma.wait_send()

    m_all = [stats_comm[s, :, 0:H] for s in range(N_DEV)]
    l_all = [stats_comm[s, :, H:2 * H] for s in range(N_DEV)]
    M = jnp.maximum(jnp.maximum(m_all[0], m_all[1]),
                    jnp.maximum(m_all[2], m_all[3]))
    L = (jnp.exp(m_all[0] - M) * l_all[0] + jnp.exp(m_all[1] - M) * l_all[1]
         + jnp.exp(m_all[2] - M) * l_all[2] + jnp.exp(m_all[3] - M) * l_all[3])
    alpha = jnp.exp(m_all[0] - M) / L
    for h in range(H):
        cs = slice(h * DH, (h + 1) * DH)
        o_buf[:, cs] = o_buf[:, cs] * alpha[:, h:h + 1]

    def rows(idx):
        return pl.ds(idx * CHUNK, CHUNK)

    def compute_chunk(idx):
        out_ref[rows(idx), :] = jnp.dot(
            o_buf[rows(idx), :], wo_ref[:, :],
            preferred_element_type=jnp.float32)

    def start_rs(t):
        cw = pltpu.make_async_remote_copy(
            src_ref=out_ref.at[rows(jnp.mod(my - t, N_DEV)), pl.ds(0, HALF)],
            dst_ref=rs_cw.at[t],
            send_sem=send_cw, recv_sem=rs_recv_cw.at[t],
            device_id=(right,), device_id_type=MESH,
        )
        ccw = pltpu.make_async_remote_copy(
            src_ref=out_ref.at[rows(jnp.mod(my + t, N_DEV)), pl.ds(HALF, HALF)],
            dst_ref=rs_ccw.at[t],
            send_sem=send_ccw, recv_sem=rs_recv_ccw.at[t],
            device_id=(left,), device_id_type=MESH,
        )
        cw.start()
        ccw.start()
        return cw, ccw

    def finish_rs(t, cw, ccw):
        cw.wait()
        ccw.wait()
        acw = jnp.mod(my - t - 1, N_DEV)
        out_ref[rows(acw), pl.ds(0, HALF)] = (
            out_ref[rows(acw), pl.ds(0, HALF)] + rs_cw[t, :, :])
        accw = jnp.mod(my + t + 1, N_DEV)
        out_ref[rows(accw), pl.ds(HALF, HALF)] = (
            out_ref[rows(accw), pl.ds(HALF, HALF)] + rs_ccw[t, :, :])

    compute_chunk(my)
    h0 = start_rs(0)
    compute_chunk(jnp.mod(my - 1, N_DEV))
    compute_chunk(jnp.mod(my + 1, N_DEV))
    finish_rs(0, *h0)
    h1 = start_rs(1)
    compute_chunk(diag)
    finish_rs(1, *h1)
    h2 = start_rs(2)
    finish_rs(2, *h2)

    for t in range(N_DEV - 1):
        fcw = jnp.mod(my + 1 - t, N_DEV)
        cw = pltpu.make_async_remote_copy(
            src_ref=out_ref.at[rows(fcw), pl.ds(0, HALF)],
            dst_ref=out_ref.at[rows(fcw), pl.ds(0, HALF)],
            send_sem=send_cw, recv_sem=ag_recv_cw.at[t],
            device_id=(right,), device_id_type=MESH,
        )
        fccw = jnp.mod(my - 1 + t, N_DEV)
        ccw = pltpu.make_async_remote_copy(
            src_ref=out_ref.at[rows(fccw), pl.ds(HALF, HALF)],
            dst_ref=out_ref.at[rows(fccw), pl.ds(HALF, HALF)],
            send_sem=send_ccw, recv_sem=ag_recv_ccw.at[t],
            device_id=(left,), device_id_type=MESH,
        )
        cw.start()
        ccw.start()
        cw.wait()
        ccw.wait()


def kernel(x, Wq, K_ext, V_ext, Wo):
    x2 = x.reshape(SQ, DM)
    k2 = K_ext.reshape(SQ, H * DH)
    v2 = V_ext.reshape(SQ, H * DH)
    out = pl.pallas_call(
        _body,
        out_shape=jax.ShapeDtypeStruct((SQ, DM), jnp.float32),
        in_specs=[
            pl.BlockSpec(memory_space=pl.ANY),
            pl.BlockSpec(memory_space=pltpu.VMEM),
            pl.BlockSpec(memory_space=pl.ANY),
            pl.BlockSpec(memory_space=pl.ANY),
            pl.BlockSpec(memory_space=pltpu.VMEM),
        ],
        out_specs=pl.BlockSpec(memory_space=pltpu.VMEM),
        scratch_shapes=[
            pltpu.VMEM((CHUNK, DM), jnp.float32),
            pltpu.VMEM((CHUNK, DM), jnp.float32),
            pltpu.VMEM((CHUNK, H * DH), jnp.float32),
            pltpu.VMEM((CHUNK, H * DH), jnp.float32),
            pltpu.VMEM((SQ, DM), jnp.float32),
            pltpu.VMEM((CHUNK, 2 * H), jnp.float32),
            pltpu.VMEM((N_DEV, SQ, 2 * H), jnp.float32),
            pltpu.SemaphoreType.DMA((3,)),
            pltpu.SemaphoreType.DMA((N_DEV - 1,)),
            pltpu.SemaphoreType.DMA((N_DEV - 1,)),
            pltpu.SemaphoreType.DMA,
            pltpu.SemaphoreType.DMA,
            pltpu.VMEM((N_DEV - 1, CHUNK, HALF), jnp.float32),
            pltpu.VMEM((N_DEV - 1, CHUNK, HALF), jnp.float32),
            pltpu.SemaphoreType.DMA((N_DEV - 1,)),
            pltpu.SemaphoreType.DMA((N_DEV - 1,)),
            pltpu.SemaphoreType.DMA((N_DEV - 1,)),
            pltpu.SemaphoreType.DMA((N_DEV - 1,)),
        ],
        compiler_params=pltpu.CompilerParams(collective_id=0),
    )(x2, Wq, k2, v2, Wo)
    return out.reshape(1, SQ, DM)


# device time: 154128 ns/iter; 1.5766x vs baseline; 1.0346x over previous
import jax
import jax.numpy as jnp
from jax import lax
from jax.experimental import pallas as pl
from jax.experimental.pallas import tpu as pltpu

N_DEV = 4
SQ = 2048
DM = 1024
H = 8
DH = 128
BLK = 64
R = 4
JB = 8
CHUNK = SQ // N_DEV
HALF = DM // 2
SCALE = 0.08838834764831843
MESH = pl.DeviceIdType.MESH


def _body(x_hbm, wq_ref, k_hbm, v_hbm, wo_ref, out_ref,
          xr, qr, kr, vr, o_buf, stats_r, stats_comm,
          dma_sems, st_send, st_recv, send_cw, send_ccw,
          rs_cw, rs_ccw, rs_recv_cw, rs_recv_ccw, ag_recv_cw, ag_recv_ccw):
    my = lax.axis_index("i")
    right = jnp.mod(my + 1, N_DEV)
    left = jnp.mod(my - 1, N_DEV)
    diag = jnp.mod(my + 2, N_DEV)

    def issue_copies(r):
        slot = r % 2
        copies = []
        for j in range(JB):
            b = r + R * j
            copies.append(pltpu.make_async_copy(
                x_hbm.at[0, pl.ds(b * BLK, BLK), :],
                xr.at[slot, pl.ds(j * BLK, BLK), :],
                dma_sems.at[slot, 0]))
            copies.append(pltpu.make_async_copy(
                k_hbm.at[0, pl.ds(b * BLK, BLK), :, :],
                kr.at[slot, pl.ds(j * BLK, BLK), :, :],
                dma_sems.at[slot, 1]))
            copies.append(pltpu.make_async_copy(
                v_hbm.at[0, pl.ds(b * BLK, BLK), :, :],
                vr.at[slot, pl.ds(j * BLK, BLK), :, :],
                dma_sems.at[slot, 2]))
        for c in copies:
            c.start()
        return copies

    pending = issue_copies(0)
    for r in range(R):
      with jax.named_scope(f"phA#r={r}"):
        slot = r % 2
        blocks = [r + R * j for j in range(JB)]
        nxt = issue_copies(r + 1) if r + 1 < R else []
        for c in pending:
            c.wait()
        pending = nxt

        qr[:, :] = jnp.dot(xr[slot], wq_ref[:, :],
                           preferred_element_type=jnp.float32)
        for h in range(H):
            cs = slice(h * DH, (h + 1) * DH)
            s = lax.dot_general(qr[:, cs], kr[slot, :, h, :],
                                (((1,), (1,)), ((), ())),
                                preferred_element_type=jnp.float32) * SCALE
            m = jnp.max(s, axis=1, keepdims=True)
            w = jnp.exp(s - m)
            l = jnp.sum(w, axis=1, keepdims=True)
            o = jnp.dot(w, vr[slot, :, h, :],
                        preferred_element_type=jnp.float32)
            for j, b in enumerate(blocks):
                o_buf[pl.ds(b * BLK, BLK), cs] = o[j * BLK:(j + 1) * BLK, :]
            stats_r[:, h:h + 1] = m
            stats_r[:, H + h:H + h + 1] = l
        for j, b in enumerate(blocks):
            stats_comm[0, pl.ds(b * BLK, BLK), :] = stats_r[pl.ds(j * BLK, BLK), :]

    with jax.named_scope("barrier"):
        barrier = pltpu.get_barrier_semaphore()
        for nbr in (left, right, diag):
            pl.semaphore_signal(barrier, inc=1, device_id=(nbr,),
                                device_id_type=MESH)
        pl.semaphore_wait(barrier, 3)

    with jax.named_scope("stats_x"):
        stat_sends = []
        for d in range(1, N_DEV):
            slot = N_DEV - d
            rdma = pltpu.make_async_remote_copy(
                src_ref=stats_comm.at[0],
                dst_ref=stats_comm.at[slot],
                send_sem=st_send.at[d - 1],
                recv_sem=st_recv.at[slot - 1],
                device_id=(jnp.mod(my + d, N_DEV),),
                device_id_type=MESH,
            )
            rdma.start()
            stat_sends.append(rdma)
        for k in range(1, N_DEV):
            pltpu.make_async_remote_copy(
                src_ref=stats_comm.at[k], dst_ref=stats_comm.at[k],
                send_sem=st_send.at[k - 1], recv_sem=st_recv.at[k - 1],
                device_id=(left,), device_id_type=MESH,
            ).wait_recv()
        for rdma in stat_sends:
            rdma.wait_send()

    with jax.named_scope("combine"):
        m_all = [stats_comm[s, :, 0:H] for s in range(N_DEV)]
        l_all = [stats_comm[s, :, H:2 * H] for s in range(N_DEV)]
        M = jnp.maximum(jnp.maximum(m_all[0], m_all[1]),
                        jnp.maximum(m_all[2], m_all[3]))
        L = (jnp.exp(m_all[0] - M) * l_all[0] + jnp.exp(m_all[1] - M) * l_all[1]
             + jnp.exp(m_all[2] - M) * l_all[2] + jnp.exp(m_all[3] - M) * l_all[3])
        alpha = jnp.exp(m_all[0] - M) / L
        for h in range(H):
            cs = slice(h * DH, (h + 1) * DH)
            o_buf[:, cs] = o_buf[:, cs] * alpha[:, h:h + 1]

    def rows(idx):
        return pl.ds(idx * CHUNK, CHUNK)

    def compute_chunk(idx, tag=""):
        with jax.named_scope(f"wo{tag}"):
            out_ref[0, rows(idx), :] = jnp.dot(
                o_buf[rows(idx), :], wo_ref[:, :],
                preferred_element_type=jnp.float32)

    def start_rs(t):
        cw = pltpu.make_async_remote_copy(
            src_ref=out_ref.at[0, rows(jnp.mod(my - t, N_DEV)), pl.ds(0, HALF)],
            dst_ref=rs_cw.at[t],
            send_sem=send_cw, recv_sem=rs_recv_cw.at[t],
            device_id=(right,), device_id_type=MESH,
        )
        ccw = pltpu.make_async_remote_copy(
            src_ref=out_ref.at[0, rows(jnp.mod(my + t, N_DEV)), pl.ds(HALF, HALF)],
            dst_ref=rs_ccw.at[t],
            send_sem=send_ccw, recv_sem=rs_recv_ccw.at[t],
            device_id=(left,), device_id_type=MESH,
        )
        cw.start()
        ccw.start()
        return cw, ccw

    def finish_rs(t, cw, ccw):
        with jax.named_scope(f"rs_wait#t={t}"):
            cw.wait()
            ccw.wait()
        with jax.named_scope(f"rs_add#t={t}"):
            acw = jnp.mod(my - t - 1, N_DEV)
            out_ref[0, rows(acw), pl.ds(0, HALF)] = (
                out_ref[0, rows(acw), pl.ds(0, HALF)] + rs_cw[t, :, :])
            accw = jnp.mod(my + t + 1, N_DEV)
            out_ref[0, rows(accw), pl.ds(HALF, HALF)] = (
                out_ref[0, rows(accw), pl.ds(HALF, HALF)] + rs_ccw[t, :, :])

    compute_chunk(my, "#c=0")
    h0 = start_rs(0)
    compute_chunk(jnp.mod(my - 1, N_DEV), "#c=1")
    compute_chunk(jnp.mod(my + 1, N_DEV), "#c=2")
    finish_rs(0, *h0)
    h1 = start_rs(1)
    compute_chunk(diag, "#c=3")
    finish_rs(1, *h1)
    h2 = start_rs(2)
    finish_rs(2, *h2)

    for t in range(N_DEV - 1):
      with jax.named_scope(f"ag#t={t}"):
        fcw = jnp.mod(my + 1 - t, N_DEV)
        cw = pltpu.make_async_remote_copy(
            src_ref=out_ref.at[0, rows(fcw), pl.ds(0, HALF)],
            dst_ref=out_ref.at[0, rows(fcw), pl.ds(0, HALF)],
            send_sem=send_cw, recv_sem=ag_recv_cw.at[t],
            device_id=(right,), device_id_type=MESH,
        )
        fccw = jnp.mod(my - 1 + t, N_DEV)
        ccw = pltpu.make_async_remote_copy(
            src_ref=out_ref.at[0, rows(fccw), pl.ds(HALF, HALF)],
            dst_ref=out_ref.at[0, rows(fccw), pl.ds(HALF, HALF)],
            send_sem=send_ccw, recv_sem=ag_recv_ccw.at[t],
            device_id=(left,), device_id_type=MESH,
        )
        cw.start()
        ccw.start()
        cw.wait()
        ccw.wait()


def kernel(x, Wq, K_ext, V_ext, Wo):
    return pl.pallas_call(
        _body,
        out_shape=jax.ShapeDtypeStruct((1, SQ, DM), jnp.float32),
        in_specs=[
            pl.BlockSpec(memory_space=pl.ANY),
            pl.BlockSpec(memory_space=pltpu.VMEM),
            pl.BlockSpec(memory_space=pl.ANY),
            pl.BlockSpec(memory_space=pl.ANY),
            pl.BlockSpec(memory_space=pltpu.VMEM),
        ],
        out_specs=pl.BlockSpec(memory_space=pltpu.VMEM),
        scratch_shapes=[
            pltpu.VMEM((2, CHUNK, DM), jnp.float32),
            pltpu.VMEM((CHUNK, DM), jnp.float32),
            pltpu.VMEM((2, CHUNK, H, DH), jnp.float32),
            pltpu.VMEM((2, CHUNK, H, DH), jnp.float32),
            pltpu.VMEM((SQ, DM), jnp.float32),
            pltpu.VMEM((CHUNK, 2 * H), jnp.float32),
            pltpu.VMEM((N_DEV, SQ, 2 * H), jnp.float32),
            pltpu.SemaphoreType.DMA((2, 3)),
            pltpu.SemaphoreType.DMA((N_DEV - 1,)),
            pltpu.SemaphoreType.DMA((N_DEV - 1,)),
            pltpu.SemaphoreType.DMA,
            pltpu.SemaphoreType.DMA,
            pltpu.VMEM((N_DEV - 1, CHUNK, HALF), jnp.float32),
            pltpu.VMEM((N_DEV - 1, CHUNK, HALF), jnp.float32),
            pltpu.SemaphoreType.DMA((N_DEV - 1,)),
            pltpu.SemaphoreType.DMA((N_DEV - 1,)),
            pltpu.SemaphoreType.DMA((N_DEV - 1,)),
            pltpu.SemaphoreType.DMA((N_DEV - 1,)),
        ],
        compiler_params=pltpu.CompilerParams(
            collective_id=0, vmem_limit_bytes=56 * 1024 * 1024),
    )(x, Wq, K_ext, V_ext, Wo)


# device time: 154105 ns/iter; 1.5768x vs baseline; 1.0001x over previous
import contextlib

import jax
import jax.numpy as jnp
from jax import lax
from jax.experimental import pallas as pl
from jax.experimental.pallas import tpu as pltpu

_PROFILE_SCOPES = False


def _scope(name):
    if _PROFILE_SCOPES:
        return jax.named_scope(name)
    return contextlib.nullcontext()

N_DEV = 4
SQ = 2048
DM = 1024
H = 8
DH = 128
BLK = 64
R = 4
JB = 8
CHUNK = SQ // N_DEV
HALF = DM // 2
SCALE = 0.08838834764831843
MESH = pl.DeviceIdType.MESH


def _body(x_hbm, wq_ref, k_hbm, v_hbm, wo_ref, out_ref,
          xr, qr, kr, vr, o_buf, stats_r, stats_comm,
          dma_sems, st_send, st_recv, send_cw, send_ccw,
          rs_cw, rs_ccw, rs_recv_cw, rs_recv_ccw, ag_recv_cw, ag_recv_ccw):
    my = lax.axis_index("i")
    right = jnp.mod(my + 1, N_DEV)
    left = jnp.mod(my - 1, N_DEV)
    diag = jnp.mod(my + 2, N_DEV)

    def issue_copies(r):
        slot = r % 2
        copies = []
        for j in range(JB):
            b = r + R * j
            copies.append(pltpu.make_async_copy(
                x_hbm.at[0, pl.ds(b * BLK, BLK), :],
                xr.at[slot, pl.ds(j * BLK, BLK), :],
                dma_sems.at[slot, 0]))
            copies.append(pltpu.make_async_copy(
                k_hbm.at[0, pl.ds(b * BLK, BLK), :, :],
                kr.at[slot, pl.ds(j * BLK, BLK), :, :],
                dma_sems.at[slot, 1]))
            copies.append(pltpu.make_async_copy(
                v_hbm.at[0, pl.ds(b * BLK, BLK), :, :],
                vr.at[slot, pl.ds(j * BLK, BLK), :, :],
                dma_sems.at[slot, 2]))
        for c in copies:
            c.start()
        return copies

    pending = issue_copies(0)
    for r in range(R):
      with _scope(f"phA#r={r}"):
        slot = r % 2
        blocks = [r + R * j for j in range(JB)]
        nxt = issue_copies(r + 1) if r + 1 < R else []
        for c in pending:
            c.wait()
        pending = nxt

        qr[:, :] = jnp.dot(xr[slot], wq_ref[:, :],
                           preferred_element_type=jnp.float32)
        for h in range(H):
            cs = slice(h * DH, (h + 1) * DH)
            s = lax.dot_general(qr[:, cs], kr[slot, :, h, :],
                                (((1,), (1,)), ((), ())),
                                preferred_element_type=jnp.float32) * SCALE
            m = jnp.max(s, axis=1, keepdims=True)
            w = jnp.exp(s - m)
            l = jnp.sum(w, axis=1, keepdims=True)
            o = jnp.dot(w, vr[slot, :, h, :],
                        preferred_element_type=jnp.float32)
            for j, b in enumerate(blocks):
                o_buf[pl.ds(b * BLK, BLK), cs] = o[j * BLK:(j + 1) * BLK, :]
            stats_r[:, h:h + 1] = m
            stats_r[:, H + h:H + h + 1] = l
        for j, b in enumerate(blocks):
            stats_comm[0, pl.ds(b * BLK, BLK), :] = stats_r[pl.ds(j * BLK, BLK), :]

    with _scope("barrier"):
        barrier = pltpu.get_barrier_semaphore()
        for nbr in (left, right, diag):
            pl.semaphore_signal(barrier, inc=1, device_id=(nbr,),
                                device_id_type=MESH)
        pl.semaphore_wait(barrier, 3)

    with _scope("stats_x"):
        stat_sends = []
        for d in range(1, N_DEV):
            slot = N_DEV - d
            rdma = pltpu.make_async_remote_copy(
                src_ref=stats_comm.at[0],
                dst_ref=stats_comm.at[slot],
                send_sem=st_send.at[d - 1],
                recv_sem=st_recv.at[slot - 1],
                device_id=(jnp.mod(my + d, N_DEV),),
                device_id_type=MESH,
            )
            rdma.start()
            stat_sends.append(rdma)
        for k in range(1, N_DEV):
            pltpu.make_async_remote_copy(
                src_ref=stats_comm.at[k], dst_ref=stats_comm.at[k],
                send_sem=st_send.at[k - 1], recv_sem=st_recv.at[k - 1],
                device_id=(left,), device_id_type=MESH,
            ).wait_recv()
        for rdma in stat_sends:
            rdma.wait_send()

    with _scope("combine"):
        m_all = [stats_comm[s, :, 0:H] for s in range(N_DEV)]
        l_all = [stats_comm[s, :, H:2 * H] for s in range(N_DEV)]
        M = jnp.maximum(jnp.maximum(m_all[0], m_all[1]),
                        jnp.maximum(m_all[2], m_all[3]))
        L = (jnp.exp(m_all[0] - M) * l_all[0] + jnp.exp(m_all[1] - M) * l_all[1]
             + jnp.exp(m_all[2] - M) * l_all[2] + jnp.exp(m_all[3] - M) * l_all[3])
        alpha = jnp.exp(m_all[0] - M) / L
        for h in range(H):
            cs = slice(h * DH, (h + 1) * DH)
            o_buf[:, cs] = o_buf[:, cs] * alpha[:, h:h + 1]

    def rows(idx):
        return pl.ds(idx * CHUNK, CHUNK)

    def compute_chunk(idx, tag=""):
        with _scope(f"wo{tag}"):
            out_ref[0, rows(idx), :] = jnp.dot(
                o_buf[rows(idx), :], wo_ref[:, :],
                preferred_element_type=jnp.float32)

    def start_rs(t):
        cw = pltpu.make_async_remote_copy(
            src_ref=out_ref.at[0, rows(jnp.mod(my - t, N_DEV)), pl.ds(0, HALF)],
            dst_ref=rs_cw.at[t],
            send_sem=send_cw, recv_sem=rs_recv_cw.at[t],
            device_id=(right,), device_id_type=MESH,
        )
        ccw = pltpu.make_async_remote_copy(
            src_ref=out_ref.at[0, rows(jnp.mod(my + t, N_DEV)), pl.ds(HALF, HALF)],
            dst_ref=rs_ccw.at[t],
            send_sem=send_ccw, recv_sem=rs_recv_ccw.at[t],
            device_id=(left,), device_id_type=MESH,
        )
        cw.start()
        ccw.start()
        return cw, ccw

    def finish_rs(t, cw, ccw):
        with _scope(f"rs_wait#t={t}"):
            cw.wait()
            ccw.wait()
        with _scope(f"rs_add#t={t}"):
            acw = jnp.mod(my - t - 1, N_DEV)
            out_ref[0, rows(acw), pl.ds(0, HALF)] = (
                out_ref[0, rows(acw), pl.ds(0, HALF)] + rs_cw[t, :, :])
            accw = jnp.mod(my + t + 1, N_DEV)
            out_ref[0, rows(accw), pl.ds(HALF, HALF)] = (
                out_ref[0, rows(accw), pl.ds(HALF, HALF)] + rs_ccw[t, :, :])

    compute_chunk(my, "#c=0")
    h0 = start_rs(0)
    compute_chunk(jnp.mod(my - 1, N_DEV), "#c=1")
    compute_chunk(jnp.mod(my + 1, N_DEV), "#c=2")
    finish_rs(0, *h0)
    h1 = start_rs(1)
    compute_chunk(diag, "#c=3")
    finish_rs(1, *h1)
    h2 = start_rs(2)
    finish_rs(2, *h2)

    for t in range(N_DEV - 1):
      with _scope(f"ag#t={t}"):
        fcw = jnp.mod(my + 1 - t, N_DEV)
        cw = pltpu.make_async_remote_copy(
            src_ref=out_ref.at[0, rows(fcw), pl.ds(0, HALF)],
            dst_ref=out_ref.at[0, rows(fcw), pl.ds(0, HALF)],
            send_sem=send_cw, recv_sem=ag_recv_cw.at[t],
            device_id=(right,), device_id_type=MESH,
        )
        fccw = jnp.mod(my - 1 + t, N_DEV)
        ccw = pltpu.make_async_remote_copy(
            src_ref=out_ref.at[0, rows(fccw), pl.ds(HALF, HALF)],
            dst_ref=out_ref.at[0, rows(fccw), pl.ds(HALF, HALF)],
            send_sem=send_ccw, recv_sem=ag_recv_ccw.at[t],
            device_id=(left,), device_id_type=MESH,
        )
        cw.start()
        ccw.start()
        cw.wait()
        ccw.wait()


def kernel(x, Wq, K_ext, V_ext, Wo):
    return pl.pallas_call(
        _body,
        out_shape=jax.ShapeDtypeStruct((1, SQ, DM), jnp.float32),
        in_specs=[
            pl.BlockSpec(memory_space=pl.ANY),
            pl.BlockSpec(memory_space=pltpu.VMEM),
            pl.BlockSpec(memory_space=pl.ANY),
            pl.BlockSpec(memory_space=pl.ANY),
            pl.BlockSpec(memory_space=pltpu.VMEM),
        ],
        out_specs=pl.BlockSpec(memory_space=pltpu.VMEM),
        scratch_shapes=[
            pltpu.VMEM((2, CHUNK, DM), jnp.float32),
            pltpu.VMEM((CHUNK, DM), jnp.float32),
            pltpu.VMEM((2, CHUNK, H, DH), jnp.float32),
            pltpu.VMEM((2, CHUNK, H, DH), jnp.float32),
            pltpu.VMEM((SQ, DM), jnp.float32),
            pltpu.VMEM((CHUNK, 2 * H), jnp.float32),
            pltpu.VMEM((N_DEV, SQ, 2 * H), jnp.float32),
            pltpu.SemaphoreType.DMA((2, 3)),
            pltpu.SemaphoreType.DMA((N_DEV - 1,)),
            pltpu.SemaphoreType.DMA((N_DEV - 1,)),
            pltpu.SemaphoreType.DMA,
            pltpu.SemaphoreType.DMA,
            pltpu.VMEM((N_DEV - 1, CHUNK, HALF), jnp.float32),
            pltpu.VMEM((N_DEV - 1, CHUNK, HALF), jnp.float32),
            pltpu.SemaphoreType.DMA((N_DEV - 1,)),
            pltpu.SemaphoreType.DMA((N_DEV - 1,)),
            pltpu.SemaphoreType.DMA((N_DEV - 1,)),
            pltpu.SemaphoreType.DMA((N_DEV - 1,)),
        ],
        compiler_params=pltpu.CompilerParams(
            collective_id=0, vmem_limit_bytes=56 * 1024 * 1024),
    )(x, Wq, K_ext, V_ext, Wo)


# device time: 76497 ns/iter; 3.1765x vs baseline; 2.0145x over previous
import contextlib
import os

import jax
import jax.numpy as jnp
from jax import lax
from jax.experimental import pallas as pl
from jax.experimental.pallas import tpu as pltpu

_PROFILE_SCOPES = False
_ABLATE = int(os.environ.get("KABL", "0"))


def _scope(name):
    if _PROFILE_SCOPES:
        return jax.named_scope(name)
    return contextlib.nullcontext()

N_DEV = 4
SQ = 2048
DM = 1024
H = 8
DH = 128
BLK = 64
R = 4
JB = 8
CHUNK = SQ // N_DEV
HALF = DM // 2
SCALE = 0.08838834764831843
MESH = pl.DeviceIdType.MESH


def _body(x_hbm, wq_ref, k_hbm, v_hbm, wo_ref, out_ref,
          xr, qr, kr, vr, o_buf, stats_r, stats_comm,
          dma_sems, st_send, st_recv, send_cw, send_ccw,
          rs_cw, rs_ccw, rs_recv_cw, rs_recv_ccw, ag_recv_cw, ag_recv_ccw):
    my = lax.axis_index("i")
    right = jnp.mod(my + 1, N_DEV)
    left = jnp.mod(my - 1, N_DEV)
    diag = jnp.mod(my + 2, N_DEV)

    def issue_copies(r):
        slot = r % 2
        copies = []
        for j in range(JB):
            b = r + R * j
            copies.append(pltpu.make_async_copy(
                x_hbm.at[0, pl.ds(b * BLK, BLK), :],
                xr.at[slot, pl.ds(j * BLK, BLK), :],
                dma_sems.at[slot, 0]))
            copies.append(pltpu.make_async_copy(
                k_hbm.at[0, pl.ds(b * BLK, BLK), :, :],
                kr.at[slot, pl.ds(j * BLK, BLK), :, :],
                dma_sems.at[slot, 1]))
            copies.append(pltpu.make_async_copy(
                v_hbm.at[0, pl.ds(b * BLK, BLK), :, :],
                vr.at[slot, pl.ds(j * BLK, BLK), :, :],
                dma_sems.at[slot, 2]))
        for c in copies:
            c.start()
        return copies

    pending = issue_copies(0)
    for r in range(R):
      with _scope(f"phA#r={r}"):
        slot = r % 2
        blocks = [r + R * j for j in range(JB)]
        nxt = issue_copies(r + 1) if r + 1 < R else []
        for c in pending:
            c.wait()
        pending = nxt

        qr[:, :] = jnp.dot(xr[slot], wq_ref[:, :],
                           preferred_element_type=jnp.float32)
        for h in range(H):
            cs = slice(h * DH, (h + 1) * DH)
            s = lax.dot_general(qr[:, cs], kr[slot, :, h, :],
                                (((1,), (1,)), ((), ())),
                                preferred_element_type=jnp.float32) * SCALE
            m = jnp.max(s, axis=1, keepdims=True)
            w = jnp.exp(s - m)
            l = jnp.sum(w, axis=1, keepdims=True)
            o = jnp.dot(w, vr[slot, :, h, :],
                        preferred_element_type=jnp.float32)
            for j, b in enumerate(blocks):
                o_buf[pl.ds(b * BLK, BLK), cs] = o[j * BLK:(j + 1) * BLK, :]
            stats_r[:, h:h + 1] = m
            stats_r[:, H + h:H + h + 1] = l
        for j, b in enumerate(blocks):
            stats_comm[0, pl.ds(b * BLK, BLK), :] = stats_r[pl.ds(j * BLK, BLK), :]

    if _ABLATE >= 3:
        out_ref[0, :, :] = o_buf[:, :]
        return

    with _scope("barrier"):
        barrier = pltpu.get_barrier_semaphore()
        for nbr in (left, right, diag):
            pl.semaphore_signal(barrier, inc=1, device_id=(nbr,),
                                device_id_type=MESH)
        pl.semaphore_wait(barrier, 3)

    with _scope("stats_x"):
        stat_sends = []
        for d in range(1, N_DEV):
            slot = N_DEV - d
            rdma = pltpu.make_async_remote_copy(
                src_ref=stats_comm.at[0],
                dst_ref=stats_comm.at[slot],
                send_sem=st_send.at[d - 1],
                recv_sem=st_recv.at[slot - 1],
                device_id=(jnp.mod(my + d, N_DEV),),
                device_id_type=MESH,
            )
            rdma.start()
            stat_sends.append(rdma)
        for k in range(1, N_DEV):
            pltpu.make_async_remote_copy(
                src_ref=stats_comm.at[k], dst_ref=stats_comm.at[k],
                send_sem=st_send.at[k - 1], recv_sem=st_recv.at[k - 1],
                device_id=(left,), device_id_type=MESH,
            ).wait_recv()
        for rdma in stat_sends:
            rdma.wait_send()

    with _scope("combine"):
        m_all = [stats_comm[s, :, 0:H] for s in range(N_DEV)]
        l_all = [stats_comm[s, :, H:2 * H] for s in range(N_DEV)]
        M = jnp.maximum(jnp.maximum(m_all[0], m_all[1]),
                        jnp.maximum(m_all[2], m_all[3]))
        L = (jnp.exp(m_all[0] - M) * l_all[0] + jnp.exp(m_all[1] - M) * l_all[1]
             + jnp.exp(m_all[2] - M) * l_all[2] + jnp.exp(m_all[3] - M) * l_all[3])
        alpha = jnp.exp(m_all[0] - M) / L
        for h in range(H):
            cs = slice(h * DH, (h + 1) * DH)
            o_buf[:, cs] = o_buf[:, cs] * alpha[:, h:h + 1]

    if _ABLATE == 2:
        out_ref[0, :, :] = o_buf[:, :]
        return

    def rows(idx):
        return pl.ds(idx * CHUNK, CHUNK)

    def compute_chunk(idx, tag=""):
        with _scope(f"wo{tag}"):
            out_ref[0, rows(idx), :] = jnp.dot(
                o_buf[rows(idx), :], wo_ref[:, :],
                preferred_element_type=jnp.float32)

    def start_rs(t):
        cw = pltpu.make_async_remote_copy(
            src_ref=out_ref.at[0, rows(jnp.mod(my - t, N_DEV)), pl.ds(0, HALF)],
            dst_ref=rs_cw.at[t],
            send_sem=send_cw, recv_sem=rs_recv_cw.at[t],
            device_id=(right,), device_id_type=MESH,
        )
        ccw = pltpu.make_async_remote_copy(
            src_ref=out_ref.at[0, rows(jnp.mod(my + t, N_DEV)), pl.ds(HALF, HALF)],
            dst_ref=rs_ccw.at[t],
            send_sem=send_ccw, recv_sem=rs_recv_ccw.at[t],
            device_id=(left,), device_id_type=MESH,
        )
        cw.start()
        ccw.start()
        return cw, ccw

    def finish_rs(t, cw, ccw):
        with _scope(f"rs_wait#t={t}"):
            cw.wait()
            ccw.wait()
        with _scope(f"rs_add#t={t}"):
            acw = jnp.mod(my - t - 1, N_DEV)
            out_ref[0, rows(acw), pl.ds(0, HALF)] = (
                out_ref[0, rows(acw), pl.ds(0, HALF)] + rs_cw[t, :, :])
            accw = jnp.mod(my + t + 1, N_DEV)
            out_ref[0, rows(accw), pl.ds(HALF, HALF)] = (
                out_ref[0, rows(accw), pl.ds(HALF, HALF)] + rs_ccw[t, :, :])

    if _ABLATE == 1:
        for c in range(N_DEV):
            compute_chunk(c, f"#c={c}")
        return

    compute_chunk(my, "#c=0")
    h0 = start_rs(0)
    compute_chunk(jnp.mod(my - 1, N_DEV), "#c=1")
    compute_chunk(jnp.mod(my + 1, N_DEV), "#c=2")
    finish_rs(0, *h0)
    h1 = start_rs(1)
    compute_chunk(diag, "#c=3")
    finish_rs(1, *h1)
    h2 = start_rs(2)
    finish_rs(2, *h2)

    for t in range(N_DEV - 1):
      with _scope(f"ag#t={t}"):
        fcw = jnp.mod(my + 1 - t, N_DEV)
        cw = pltpu.make_async_remote_copy(
            src_ref=out_ref.at[0, rows(fcw), pl.ds(0, HALF)],
            dst_ref=out_ref.at[0, rows(fcw), pl.ds(0, HALF)],
            send_sem=send_cw, recv_sem=ag_recv_cw.at[t],
            device_id=(right,), device_id_type=MESH,
        )
        fccw = jnp.mod(my - 1 + t, N_DEV)
        ccw = pltpu.make_async_remote_copy(
            src_ref=out_ref.at[0, rows(fccw), pl.ds(HALF, HALF)],
            dst_ref=out_ref.at[0, rows(fccw), pl.ds(HALF, HALF)],
            send_sem=send_ccw, recv_sem=ag_recv_ccw.at[t],
            device_id=(left,), device_id_type=MESH,
        )
        cw.start()
        ccw.start()
        cw.wait()
        ccw.wait()


def kernel(x, Wq, K_ext, V_ext, Wo):
    return pl.pallas_call(
        _body,
        out_shape=jax.ShapeDtypeStruct((1, SQ, DM), jnp.float32),
        in_specs=[
            pl.BlockSpec(memory_space=pl.ANY),
            pl.BlockSpec(memory_space=pltpu.VMEM),
            pl.BlockSpec(memory_space=pl.ANY),
            pl.BlockSpec(memory_space=pl.ANY),
            pl.BlockSpec(memory_space=pltpu.VMEM),
        ],
        out_specs=pl.BlockSpec(memory_space=pltpu.VMEM),
        scratch_shapes=[
            pltpu.VMEM((2, CHUNK, DM), jnp.float32),
            pltpu.VMEM((CHUNK, DM), jnp.float32),
            pltpu.VMEM((2, CHUNK, H, DH), jnp.float32),
            pltpu.VMEM((2, CHUNK, H, DH), jnp.float32),
            pltpu.VMEM((SQ, DM), jnp.float32),
            pltpu.VMEM((CHUNK, 2 * H), jnp.float32),
            pltpu.VMEM((N_DEV, SQ, 2 * H), jnp.float32),
            pltpu.SemaphoreType.DMA((2, 3)),
            pltpu.SemaphoreType.DMA((N_DEV - 1,)),
            pltpu.SemaphoreType.DMA((N_DEV - 1,)),
            pltpu.SemaphoreType.DMA,
            pltpu.SemaphoreType.DMA,
            pltpu.VMEM((N_DEV - 1, CHUNK, HALF), jnp.float32),
            pltpu.VMEM((N_DEV - 1, CHUNK, HALF), jnp.float32),
            pltpu.SemaphoreType.DMA((N_DEV - 1,)),
            pltpu.SemaphoreType.DMA((N_DEV - 1,)),
            pltpu.SemaphoreType.DMA((N_DEV - 1,)),
            pltpu.SemaphoreType.DMA((N_DEV - 1,)),
        ],
        compiler_params=pltpu.CompilerParams(
            collective_id=0, vmem_limit_bytes=56 * 1024 * 1024),
    )(x, Wq, K_ext, V_ext, Wo)
